# baseline (speedup 1.0000x reference)
import functools

import jax
import jax.numpy as jnp
from jax import lax
from jax.experimental import pallas as pl
from jax.experimental.pallas import tpu as pltpu
from jax.experimental.pallas import tpu_sc as plsc

_NC = 2
_NS = 16
_NW = _NC * _NS
_LANES = 16
_NBUF = 4
_LOOK = 2


_NSPLIT = 2


def kernel(input_ids, position_ids, subword_table, m1_table, m2_table):
    B, L = input_ids.shape
    D = subword_table.shape[1]
    n_chunks = B // (_NW * _NSPLIT)
    assert B % (_NW * _NSPLIT) == 0 and L % 8 == 0 and D % _LANES == 0
    assert n_chunks % _NBUF == 0 and n_chunks >= 3 * _NBUF

    pm1 = jnp.mod(position_ids, 47).astype(jnp.int32)
    pm2 = jnp.mod(position_ids, 11).astype(jnp.int32)
    idx = input_ids.astype(jnp.int32).reshape(_NSPLIT, _NW, n_chunks, L)

    mesh = plsc.VectorSubcoreMesh(core_axis_name="c", subcore_axis_name="s")

    @functools.partial(
        pl.kernel,
        out_type=jax.ShapeDtypeStruct((B // _NSPLIT, L, D), jnp.float32),
        mesh=mesh,
        compiler_params=pltpu.CompilerParams(use_tc_tiling_on_sc=False),
        scratch_types=[
            pltpu.VMEM((n_chunks, L), jnp.int32),
            pltpu.VMEM((_NBUF * L, D), jnp.float32),
            pltpu.VMEM((L, D), jnp.float32),
            pltpu.VMEM((L, D), jnp.float32),
            pltpu.VMEM((L,), jnp.int32),
            pltpu.VMEM((L,), jnp.int32),
            [pltpu.SemaphoreType.DMA] * _NBUF,
            [pltpu.SemaphoreType.DMA] * _NBUF,
        ],
    )
    def k(idx_hbm, pm1_hbm, pm2_hbm, table_hbm, m1_hbm, m2_hbm,
          out_hbm, idx_v, rows, pos_v, tmp_v, pm1_v, pm2_v, gsem, osem):
        wid = lax.axis_index("s") * _NC + lax.axis_index("c")
        b0 = wid * n_chunks

        def rows_sl(b):
            return rows.at[pl.ds(b * L, L), :]

        pltpu.sync_copy(pm1_hbm, pm1_v)
        pltpu.sync_copy(pm2_hbm, pm2_v)
        pltpu.async_copy(m1_hbm.at[pm1_v], tmp_v, gsem[0]).wait()
        pltpu.async_copy(m2_hbm.at[pm2_v], pos_v, gsem[0]).wait()

        @pl.loop(0, L)
        def _(r):
            for c in range(D // _LANES):
                sl = pl.ds(c * _LANES, _LANES)
                plsc.addupdate(pos_v.at[r, sl], tmp_v[r, sl])

        pltpu.sync_copy(idx_hbm.at[wid], idx_v)

        def issue_gather(t, b):
            pltpu.async_copy(table_hbm.at[idx_v.at[t]], rows_sl(b), gsem[b])

        def wait_gather(b):
            pltpu.make_async_copy(
                table_hbm.at[idx_v.at[0]], rows_sl(b), gsem[b]).wait()

        def add_pos(t, b):
            @pl.loop(0, L, step=8)
            def _(j):
                for jj in range(8):
                    for c in range(D // _LANES):
                        sl = pl.ds(c * _LANES, _LANES)
                        plsc.addupdate(rows.at[b * L + j + jj, sl],
                                       pos_v[j + jj, sl])

        def issue_out(t, b):
            pltpu.async_copy(rows_sl(b), out_hbm.at[b0 + t], osem[b])

        def wait_out(b):
            pltpu.make_async_copy(rows_sl(b), out_hbm.at[b0], osem[b]).wait()

        for b in range(_LOOK):
            issue_gather(b, b)
        for t in range(_NBUF):
            b = t % _NBUF
            wait_gather(b)
            add_pos(t, b)
            issue_out(t, b)
            b2 = (t + _LOOK) % _NBUF
            if t + _LOOK >= _NBUF:
                wait_out(b2)
            issue_gather(t + _LOOK, b2)

        @pl.loop(1, n_chunks // _NBUF - 1)
        def _(i):
            for b in range(_NBUF):
                t = _NBUF * i + b
                wait_gather(b)
                add_pos(t, b)
                issue_out(t, b)
                b2 = (b + _LOOK) % _NBUF
                wait_out(b2)
                issue_gather(t + _LOOK, b2)

        for t in range(n_chunks - _NBUF, n_chunks):
            b = t % _NBUF
            wait_gather(b)
            add_pos(t, b)
            issue_out(t, b)
            if t + _LOOK < n_chunks:
                b2 = (b + _LOOK) % _NBUF
                wait_out(b2)
                issue_gather(t + _LOOK, b2)

        for b in range(_NBUF):
            wait_out(b)

    parts = [k(idx[s], pm1, pm2, subword_table, m1_table, m2_table)
             for s in range(_NSPLIT)]
    return jnp.concatenate(parts, axis=0)


# speedup vs baseline: 1.1996x; 1.1996x over previous
import functools

import jax
import jax.numpy as jnp
from jax import lax
from jax.experimental import pallas as pl
from jax.experimental.pallas import tpu as pltpu
from jax.experimental.pallas import tpu_sc as plsc

_NC = 2
_NS = 16
_NW = _NC * _NS
_LANES = 16
_NBUF = 4
_LOOK = 2


_NSPLIT = 1


def kernel(input_ids, position_ids, subword_table, m1_table, m2_table):
    B, L = input_ids.shape
    D = subword_table.shape[1]
    n_chunks = B // (_NW * _NSPLIT)
    assert B % (_NW * _NSPLIT) == 0 and L % 8 == 0 and D % _LANES == 0
    assert n_chunks % _NBUF == 0 and n_chunks >= 3 * _NBUF

    pm1 = jnp.mod(position_ids, 47).astype(jnp.int32)
    pm2 = jnp.mod(position_ids, 11).astype(jnp.int32)
    idx = input_ids.astype(jnp.int32).reshape(_NSPLIT, _NW, n_chunks, L)

    mesh = plsc.VectorSubcoreMesh(core_axis_name="c", subcore_axis_name="s")

    @functools.partial(
        pl.kernel,
        out_type=jax.ShapeDtypeStruct((B // _NSPLIT, L, D), jnp.float32),
        mesh=mesh,
        compiler_params=pltpu.CompilerParams(use_tc_tiling_on_sc=False),
        scratch_types=[
            pltpu.VMEM((n_chunks, L), jnp.int32),
            pltpu.VMEM((_NBUF * L, D), jnp.float32),
            pltpu.VMEM((L, D), jnp.float32),
            pltpu.VMEM((L, D), jnp.float32),
            pltpu.VMEM((L,), jnp.int32),
            pltpu.VMEM((L,), jnp.int32),
            [pltpu.SemaphoreType.DMA] * _NBUF,
            [pltpu.SemaphoreType.DMA] * _NBUF,
        ],
    )
    def k(idx_hbm, pm1_hbm, pm2_hbm, table_hbm, m1_hbm, m2_hbm,
          out_hbm, idx_v, rows, pos_v, tmp_v, pm1_v, pm2_v, gsem, osem):
        wid = lax.axis_index("s") * _NC + lax.axis_index("c")
        b0 = wid * n_chunks

        def rows_sl(b):
            return rows.at[pl.ds(b * L, L), :]

        pltpu.sync_copy(pm1_hbm, pm1_v)
        pltpu.sync_copy(pm2_hbm, pm2_v)
        pltpu.async_copy(m1_hbm.at[pm1_v], tmp_v, gsem[0]).wait()
        pltpu.async_copy(m2_hbm.at[pm2_v], pos_v, gsem[0]).wait()

        @pl.loop(0, L)
        def _(r):
            for c in range(D // _LANES):
                sl = pl.ds(c * _LANES, _LANES)
                plsc.addupdate(pos_v.at[r, sl], tmp_v[r, sl])

        pltpu.sync_copy(idx_hbm.at[wid], idx_v)

        def issue_gather(t, b):
            pltpu.async_copy(table_hbm.at[idx_v.at[t]], rows_sl(b), gsem[b])

        def wait_gather(b):
            pltpu.make_async_copy(
                table_hbm.at[idx_v.at[0]], rows_sl(b), gsem[b]).wait()

        def add_pos(t, b):
            @pl.loop(0, L, step=8)
            def _(j):
                for jj in range(8):
                    for c in range(D // _LANES):
                        sl = pl.ds(c * _LANES, _LANES)
                        plsc.addupdate(rows.at[b * L + j + jj, sl],
                                       pos_v[j + jj, sl])

        def issue_out(t, b):
            pltpu.async_copy(rows_sl(b), out_hbm.at[b0 + t], osem[b])

        def wait_out(b):
            pltpu.make_async_copy(rows_sl(b), out_hbm.at[b0], osem[b]).wait()

        for b in range(_LOOK):
            issue_gather(b, b)
        for t in range(_NBUF):
            b = t % _NBUF
            wait_gather(b)
            add_pos(t, b)
            issue_out(t, b)
            b2 = (t + _LOOK) % _NBUF
            if t + _LOOK >= _NBUF:
                wait_out(b2)
            issue_gather(t + _LOOK, b2)

        @pl.loop(1, n_chunks // _NBUF - 1)
        def _(i):
            for b in range(_NBUF):
                t = _NBUF * i + b
                wait_gather(b)
                add_pos(t, b)
                issue_out(t, b)
                b2 = (b + _LOOK) % _NBUF
                wait_out(b2)
                issue_gather(t + _LOOK, b2)

        for t in range(n_chunks - _NBUF, n_chunks):
            b = t % _NBUF
            wait_gather(b)
            add_pos(t, b)
            issue_out(t, b)
            if t + _LOOK < n_chunks:
                b2 = (b + _LOOK) % _NBUF
                wait_out(b2)
                issue_gather(t + _LOOK, b2)

        for b in range(_NBUF):
            wait_out(b)

    parts = [k(idx[s], pm1, pm2, subword_table, m1_table, m2_table)
             for s in range(_NSPLIT)]
    return jnp.concatenate(parts, axis=0)


# speedup vs baseline: 1.2148x; 1.0126x over previous
import functools

import jax
import jax.numpy as jnp
from jax import lax
from jax.experimental import pallas as pl
from jax.experimental.pallas import tpu as pltpu
from jax.experimental.pallas import tpu_sc as plsc

_NC = 2
_NS = 16
_NW = _NC * _NS
_LANES = 16
_NBUF = 4
_LOOK = 3


_NSPLIT = 1


def kernel(input_ids, position_ids, subword_table, m1_table, m2_table):
    B, L = input_ids.shape
    D = subword_table.shape[1]
    n_chunks = B // (_NW * _NSPLIT)
    assert B % (_NW * _NSPLIT) == 0 and L % 8 == 0 and D % _LANES == 0
    assert n_chunks % _NBUF == 0 and n_chunks >= 3 * _NBUF

    pm1 = jnp.mod(position_ids, 47).astype(jnp.int32)
    pm2 = jnp.mod(position_ids, 11).astype(jnp.int32)
    idx = input_ids.astype(jnp.int32).reshape(_NSPLIT, _NW, n_chunks, L)

    mesh = plsc.VectorSubcoreMesh(core_axis_name="c", subcore_axis_name="s")

    @functools.partial(
        pl.kernel,
        out_type=jax.ShapeDtypeStruct((B // _NSPLIT, L, D), jnp.float32),
        mesh=mesh,
        compiler_params=pltpu.CompilerParams(use_tc_tiling_on_sc=False),
        scratch_types=[
            pltpu.VMEM((n_chunks, L), jnp.int32),
            pltpu.VMEM((_NBUF * L, D), jnp.float32),
            pltpu.VMEM((L, D), jnp.float32),
            pltpu.VMEM((L, D), jnp.float32),
            pltpu.VMEM((L,), jnp.int32),
            pltpu.VMEM((L,), jnp.int32),
            [pltpu.SemaphoreType.DMA] * _NBUF,
            [pltpu.SemaphoreType.DMA] * _NBUF,
        ],
    )
    def k(idx_hbm, pm1_hbm, pm2_hbm, table_hbm, m1_hbm, m2_hbm,
          out_hbm, idx_v, rows, pos_v, tmp_v, pm1_v, pm2_v, gsem, osem):
        wid = lax.axis_index("s") * _NC + lax.axis_index("c")
        b0 = wid * n_chunks

        def rows_sl(b):
            return rows.at[pl.ds(b * L, L), :]

        pltpu.sync_copy(pm1_hbm, pm1_v)
        pltpu.sync_copy(pm2_hbm, pm2_v)
        pltpu.async_copy(m1_hbm.at[pm1_v], tmp_v, gsem[0]).wait()
        pltpu.async_copy(m2_hbm.at[pm2_v], pos_v, gsem[0]).wait()

        @pl.loop(0, L)
        def _(r):
            for c in range(D // _LANES):
                sl = pl.ds(c * _LANES, _LANES)
                plsc.addupdate(pos_v.at[r, sl], tmp_v[r, sl])

        pltpu.sync_copy(idx_hbm.at[wid], idx_v)

        def issue_gather(t, b):
            pltpu.async_copy(table_hbm.at[idx_v.at[t]], rows_sl(b), gsem[b])

        def wait_gather(b):
            pltpu.make_async_copy(
                table_hbm.at[idx_v.at[0]], rows_sl(b), gsem[b]).wait()

        def add_pos(t, b):
            @pl.loop(0, L, step=8)
            def _(j):
                for jj in range(8):
                    for c in range(D // _LANES):
                        sl = pl.ds(c * _LANES, _LANES)
                        plsc.addupdate(rows.at[b * L + j + jj, sl],
                                       pos_v[j + jj, sl])

        def issue_out(t, b):
            pltpu.async_copy(rows_sl(b), out_hbm.at[b0 + t], osem[b])

        def wait_out(b):
            pltpu.make_async_copy(rows_sl(b), out_hbm.at[b0], osem[b]).wait()

        for b in range(_LOOK):
            issue_gather(b, b)
        for t in range(_NBUF):
            b = t % _NBUF
            wait_gather(b)
            add_pos(t, b)
            issue_out(t, b)
            b2 = (t + _LOOK) % _NBUF
            if t + _LOOK >= _NBUF:
                wait_out(b2)
            issue_gather(t + _LOOK, b2)

        @pl.loop(1, n_chunks // _NBUF - 1)
        def _(i):
            for b in range(_NBUF):
                t = _NBUF * i + b
                wait_gather(b)
                add_pos(t, b)
                issue_out(t, b)
                b2 = (b + _LOOK) % _NBUF
                wait_out(b2)
                issue_gather(t + _LOOK, b2)

        for t in range(n_chunks - _NBUF, n_chunks):
            b = t % _NBUF
            wait_gather(b)
            add_pos(t, b)
            issue_out(t, b)
            if t + _LOOK < n_chunks:
                b2 = (b + _LOOK) % _NBUF
                wait_out(b2)
                issue_gather(t + _LOOK, b2)

        for b in range(_NBUF):
            wait_out(b)

    parts = [k(idx[s], pm1, pm2, subword_table, m1_table, m2_table)
             for s in range(_NSPLIT)]
    return jnp.concatenate(parts, axis=0)


# speedup vs baseline: 1.6010x; 1.3179x over previous
import functools

import jax
import jax.numpy as jnp
from jax import lax
from jax.experimental import pallas as pl
from jax.experimental.pallas import tpu as pltpu
from jax.experimental.pallas import tpu_sc as plsc

_NC = 2
_NS = 16
_NW = _NC * _NS
_LANES = 16
_NBUF = 4
_LOOK = 3


_NSPLIT = 1


def kernel(input_ids, position_ids, subword_table, m1_table, m2_table):
    B, L = input_ids.shape
    D = subword_table.shape[1]
    n_chunks = B // (_NW * _NSPLIT)
    assert B % (_NW * _NSPLIT) == 0 and L % 8 == 0 and D % _LANES == 0
    assert n_chunks % _NBUF == 0 and n_chunks >= 3 * _NBUF

    pm1 = jnp.mod(position_ids, 47).astype(jnp.int32)
    pm2 = jnp.mod(position_ids, 11).astype(jnp.int32)
    idx = input_ids.astype(jnp.int32).reshape(_NSPLIT, _NW, n_chunks, L)

    mesh = plsc.VectorSubcoreMesh(core_axis_name="c", subcore_axis_name="s")

    @functools.partial(
        pl.kernel,
        out_type=jax.ShapeDtypeStruct((B // _NSPLIT, L, 128), jnp.float32),
        mesh=mesh,
        compiler_params=pltpu.CompilerParams(use_tc_tiling_on_sc=False),
        scratch_types=[
            pltpu.VMEM((n_chunks, L), jnp.int32),
            pltpu.VMEM((_NBUF * L, D), jnp.float32),
            pltpu.VMEM((L, D), jnp.float32),
            pltpu.VMEM((L, D), jnp.float32),
            pltpu.VMEM((L,), jnp.int32),
            pltpu.VMEM((L,), jnp.int32),
            [pltpu.SemaphoreType.DMA] * _NBUF,
            [pltpu.SemaphoreType.DMA] * _NBUF,
        ],
    )
    def k(idx_hbm, pm1_hbm, pm2_hbm, table_hbm, m1_hbm, m2_hbm,
          out_hbm, idx_v, rows, pos_v, tmp_v, pm1_v, pm2_v, gsem, osem):
        wid = lax.axis_index("s") * _NC + lax.axis_index("c")
        b0 = wid * n_chunks

        def rows_sl(b):
            return rows.at[pl.ds(b * L, L), :]

        pltpu.sync_copy(pm1_hbm, pm1_v)
        pltpu.sync_copy(pm2_hbm, pm2_v)
        pltpu.async_copy(m1_hbm.at[pm1_v], tmp_v, gsem[0]).wait()
        pltpu.async_copy(m2_hbm.at[pm2_v], pos_v, gsem[0]).wait()

        @pl.loop(0, L)
        def _(r):
            for c in range(D // _LANES):
                sl = pl.ds(c * _LANES, _LANES)
                plsc.addupdate(pos_v.at[r, sl], tmp_v[r, sl])

        pltpu.sync_copy(idx_hbm.at[wid], idx_v)

        def issue_gather(t, b):
            pltpu.async_copy(table_hbm.at[idx_v.at[t]], rows_sl(b), gsem[b])

        def wait_gather(b):
            pltpu.make_async_copy(
                table_hbm.at[idx_v.at[0]], rows_sl(b), gsem[b]).wait()

        def add_pos(t, b):
            @pl.loop(0, L, step=8)
            def _(j):
                for jj in range(8):
                    for c in range(D // _LANES):
                        sl = pl.ds(c * _LANES, _LANES)
                        plsc.addupdate(rows.at[b * L + j + jj, sl],
                                       pos_v[j + jj, sl])

        def issue_out(t, b):
            pltpu.async_copy(rows_sl(b), out_hbm.at[b0 + t, :, pl.ds(0, D)],
                             osem[b])

        def wait_out(b):
            pltpu.make_async_copy(rows_sl(b), out_hbm.at[b0, :, pl.ds(0, D)],
                                  osem[b]).wait()

        for b in range(_LOOK):
            issue_gather(b, b)
        for t in range(_NBUF):
            b = t % _NBUF
            wait_gather(b)
            add_pos(t, b)
            issue_out(t, b)
            b2 = (t + _LOOK) % _NBUF
            if t + _LOOK >= _NBUF:
                wait_out(b2)
            issue_gather(t + _LOOK, b2)

        @pl.loop(1, n_chunks // _NBUF - 1)
        def _(i):
            for b in range(_NBUF):
                t = _NBUF * i + b
                wait_gather(b)
                add_pos(t, b)
                issue_out(t, b)
                b2 = (b + _LOOK) % _NBUF
                wait_out(b2)
                issue_gather(t + _LOOK, b2)

        for t in range(n_chunks - _NBUF, n_chunks):
            b = t % _NBUF
            wait_gather(b)
            add_pos(t, b)
            issue_out(t, b)
            if t + _LOOK < n_chunks:
                b2 = (b + _LOOK) % _NBUF
                wait_out(b2)
                issue_gather(t + _LOOK, b2)

        for b in range(_NBUF):
            wait_out(b)

    parts = [k(idx[s], pm1, pm2, subword_table, m1_table, m2_table)
             for s in range(_NSPLIT)]
    return jnp.concatenate(parts, axis=0)[:, :, :D]
